# Initial kernel scaffold; baseline (speedup 1.0000x reference)
import jax
import jax.numpy as jnp
from jax.experimental import pallas as pl


def kernel(input, timestamp, train, W, b):
    raise NotImplementedError("write your pallas kernel here")


# speedup vs baseline: 21.8439x; 21.8439x over previous
import functools

import jax
import jax.numpy as jnp
from jax import lax
from jax.experimental import pallas as pl
from jax.experimental.pallas import tpu as pltpu
from jax.experimental.pallas import tpu_sc as plsc

N_TIME_INTERVAL = 100
PER_TIME = 10000.0
OUTPUT_DIM = 8

B = 4096
L = 200

NUM_CORES = 2
NUM_SUBCORES = 16
NW = NUM_CORES * NUM_SUBCORES

ROWS_PER_WORKER = B // NW
ROWS_PER_GROUP = 8
GROUPS = ROWS_PER_WORKER // ROWS_PER_GROUP

TS_ROW = L + 1
OUT_ROW = L * OUTPUT_DIM
TS_GROUP = ROWS_PER_GROUP * TS_ROW
OUT_GROUP = ROWS_PER_GROUP * OUT_ROW
NVEC = 13

_mesh = plsc.VectorSubcoreMesh(core_axis_name="c", subcore_axis_name="s")


@functools.partial(
    pl.kernel,
    out_type=jax.ShapeDtypeStruct((B * OUT_ROW,), jnp.float32),
    mesh=_mesh,
    scratch_types=[
        pltpu.VMEM((TS_GROUP + 8,), jnp.int32),
        pltpu.VMEM((OUT_GROUP + 64,), jnp.float32),
        pltpu.VMEM((N_TIME_INTERVAL * OUTPUT_DIM,), jnp.float32),
    ],
    compiler_params=pltpu.CompilerParams(needs_layout_passes=False),
)
def _time_encode(ts_hbm, table_hbm, out_hbm, ts_v, out_v, table_v):
    wid = lax.axis_index("s") * NUM_CORES + lax.axis_index("c")
    pltpu.sync_copy(table_hbm, table_v)

    iota = lax.iota(jnp.int32, 16)
    iota8 = iota * 8

    def per_row(r, _):
        ts_off = r * TS_ROW
        out_off = r * OUT_ROW
        for v in range(NVEC):
            l0 = v * 16
            t2 = ts_v[pl.ds(ts_off + l0, 16)]
            t1 = ts_v[pl.ds(ts_off + l0 + 1, 16)]
            q = (t1 - t2).astype(jnp.float32) / PER_TIME
            idx = q.astype(jnp.int32)
            idx = jnp.minimum(jnp.maximum(idx, 0), N_TIME_INTERVAL - 1)
            pos = idx * OUTPUT_DIM
            epos = iota8 + (out_off + l0 * OUTPUT_DIM)
            for k in range(OUTPUT_DIM):
                vals = plsc.load_gather(table_v, [pos + k])
                plsc.store_scatter(out_v, [epos + k], vals)
        return ()

    def per_group(g, _):
        base_row = wid * ROWS_PER_WORKER + g * ROWS_PER_GROUP
        pltpu.sync_copy(
            ts_hbm.at[pl.ds(base_row * TS_ROW, TS_GROUP)],
            ts_v.at[pl.ds(0, TS_GROUP)],
        )
        lax.fori_loop(0, ROWS_PER_GROUP, per_row, (), unroll=1)
        pltpu.sync_copy(
            out_v.at[pl.ds(0, OUT_GROUP)],
            out_hbm.at[pl.ds(base_row * OUT_ROW, OUT_GROUP)],
        )
        return ()

    lax.fori_loop(0, GROUPS, per_group, (), unroll=1)


def kernel(input, timestamp, train, W, b):
    del input, train
    table = (W.T + b[None, :]).astype(jnp.float32).reshape(-1)
    ts_flat = timestamp.astype(jnp.int32).reshape(-1)
    out = _time_encode(ts_flat, table)
    return (out.reshape(B, L, OUTPUT_DIM), timestamp[:, :-1])
